# Initial kernel scaffold; baseline (speedup 1.0000x reference)
import jax
import jax.numpy as jnp
from jax.experimental import pallas as pl


def kernel(input, codebook):
    raise NotImplementedError("write your pallas kernel here")


# speedup vs baseline: 1.0937x; 1.0937x over previous
import functools

import jax
import jax.numpy as jnp
from jax.experimental import pallas as pl

K = 2048
D = 256
BLK = 512


def _vq_block_kernel(z_ref, cb_ref, q_ref, ids_ref, loss_ref):
    z = z_ref[...]
    cb = cb_ref[...]

    mm = jax.lax.dot_general(
        z, cb, (((1,), (1,)), ((), ())),
        preferred_element_type=jnp.float32,
    )
    z2 = jnp.sum(z * z, axis=1, keepdims=True)
    c2 = jnp.sum(cb * cb, axis=1)
    dist = (z2 - 2.0 * mm) + c2[None, :]

    minval = jnp.min(dist, axis=1, keepdims=True)
    iota = jax.lax.broadcasted_iota(jnp.int32, (BLK, K), 1)
    ids = jnp.min(jnp.where(dist == minval, iota, K), axis=1)

    onehot = (iota == ids[:, None]).astype(jnp.bfloat16)
    cb_hi = cb.astype(jnp.bfloat16)
    cb_lo = (cb - cb_hi.astype(jnp.float32)).astype(jnp.bfloat16)
    q = (jax.lax.dot_general(onehot, cb_hi, (((1,), (0,)), ((), ())),
                             preferred_element_type=jnp.float32)
         + jax.lax.dot_general(onehot, cb_lo, (((1,), (0,)), ((), ())),
                               preferred_element_type=jnp.float32))

    q_ref[...] = q
    ids_ref[...] = ids.reshape(1, 1, BLK)
    loss_ref[...] = (q - z) ** 2


@functools.partial(jax.jit, static_argnames=())
def kernel(input, codebook):
    B, T, _ = input.shape
    n_tok = B * T
    n_blk = n_tok // BLK
    z = input.reshape(n_tok, D)

    q, ids3, loss = pl.pallas_call(
        _vq_block_kernel,
        grid=(n_blk,),
        in_specs=[
            pl.BlockSpec((BLK, D), lambda i: (i, 0)),
            pl.BlockSpec((K, D), lambda i: (0, 0)),
        ],
        out_specs=[
            pl.BlockSpec((BLK, D), lambda i: (i, 0)),
            pl.BlockSpec((1, 1, BLK), lambda i: (i, 0, 0)),
            pl.BlockSpec((BLK, D), lambda i: (i, 0)),
        ],
        out_shape=[
            jax.ShapeDtypeStruct((n_tok, D), jnp.float32),
            jax.ShapeDtypeStruct((n_blk, 1, BLK), jnp.int32),
            jax.ShapeDtypeStruct((n_tok, D), jnp.float32),
        ],
    )(z, codebook)

    q = q.reshape(B, T, D)
    ids = ids3.reshape(B, T)
    loss = loss.reshape(B, T, D)
    return (q, ids, loss, loss)


# speedup vs baseline: 1.1092x; 1.0141x over previous
import functools

import jax
import jax.numpy as jnp
from jax.experimental import pallas as pl
from jax.experimental.pallas import tpu as pltpu

K = 2048
D = 256
BLK = 512


def _vq_block_kernel(z_ref, cb_ref, q_ref, ids_ref, loss_ref,
                     c2_ref, cbhi_ref, cblo_ref):
    @pl.when(pl.program_id(0) == 0)
    def _prologue():
        cb0 = cb_ref[...]
        c2_ref[...] = jnp.sum(cb0 * cb0, axis=1)[None, :]
        hi = cb0.astype(jnp.bfloat16)
        cbhi_ref[...] = hi
        cblo_ref[...] = (cb0 - hi.astype(jnp.float32)).astype(jnp.bfloat16)

    z = z_ref[...]
    cb = cb_ref[...]

    mm = jax.lax.dot_general(
        z, cb, (((1,), (1,)), ((), ())),
        preferred_element_type=jnp.float32,
    )
    z2 = jnp.sum(z * z, axis=1, keepdims=True)
    dist = (z2 - 2.0 * mm) + c2_ref[...]

    minval = jnp.min(dist, axis=1, keepdims=True)
    iota = jax.lax.broadcasted_iota(jnp.int32, (BLK, K), 1)
    ids = jnp.min(jnp.where(dist == minval, iota, K), axis=1)

    onehot = (iota == ids[:, None]).astype(jnp.bfloat16)
    q = (jax.lax.dot_general(onehot, cbhi_ref[...], (((1,), (0,)), ((), ())),
                             preferred_element_type=jnp.float32)
         + jax.lax.dot_general(onehot, cblo_ref[...], (((1,), (0,)), ((), ())),
                               preferred_element_type=jnp.float32))

    q_ref[...] = q
    ids_ref[...] = ids.reshape(1, 1, BLK)
    loss_ref[...] = (q - z) ** 2


@functools.partial(jax.jit, static_argnames=())
def kernel(input, codebook):
    B, T, _ = input.shape
    n_tok = B * T
    n_blk = n_tok // BLK
    z = input.reshape(n_tok, D)

    q, ids3, loss = pl.pallas_call(
        _vq_block_kernel,
        grid=(n_blk,),
        in_specs=[
            pl.BlockSpec((BLK, D), lambda i: (i, 0)),
            pl.BlockSpec((K, D), lambda i: (0, 0)),
        ],
        out_specs=[
            pl.BlockSpec((BLK, D), lambda i: (i, 0)),
            pl.BlockSpec((1, 1, BLK), lambda i: (i, 0, 0)),
            pl.BlockSpec((BLK, D), lambda i: (i, 0)),
        ],
        out_shape=[
            jax.ShapeDtypeStruct((n_tok, D), jnp.float32),
            jax.ShapeDtypeStruct((n_blk, 1, BLK), jnp.int32),
            jax.ShapeDtypeStruct((n_tok, D), jnp.float32),
        ],
        scratch_shapes=[
            pltpu.VMEM((1, K), jnp.float32),
            pltpu.VMEM((K, D), jnp.bfloat16),
            pltpu.VMEM((K, D), jnp.bfloat16),
        ],
    )(z, codebook)

    q = q.reshape(B, T, D)
    ids = ids3.reshape(B, T)
    loss = loss.reshape(B, T, D)
    return (q, ids, loss, loss)


# speedup vs baseline: 1.1959x; 1.0782x over previous
import functools

import jax
import jax.numpy as jnp
from jax.experimental import pallas as pl
from jax.experimental.pallas import tpu as pltpu

K = 2048
D = 256
BLK = 1024


def _vq_block_kernel(z_ref, cb_ref, q_ref, ids_ref, loss_ref,
                     c2_ref, cbhi_ref, cblo_ref):
    @pl.when(pl.program_id(0) == 0)
    def _prologue():
        cb0 = cb_ref[...]
        c2_ref[...] = jnp.sum(cb0 * cb0, axis=1)[None, :]
        hi = cb0.astype(jnp.bfloat16)
        cbhi_ref[...] = hi
        cblo_ref[...] = (cb0 - hi.astype(jnp.float32)).astype(jnp.bfloat16)

    z = z_ref[...]
    cb = cb_ref[...]

    mm = jax.lax.dot_general(
        z, cb, (((1,), (1,)), ((), ())),
        preferred_element_type=jnp.float32,
    )
    z2 = jnp.sum(z * z, axis=1, keepdims=True)
    dist = (z2 - 2.0 * mm) + c2_ref[...]

    minval = jnp.min(dist, axis=1, keepdims=True)
    iota = jax.lax.broadcasted_iota(jnp.int32, (BLK, K), 1)
    ids = jnp.min(jnp.where(dist == minval, iota, K), axis=1)

    onehot = (iota == ids[:, None]).astype(jnp.bfloat16)
    q = (jax.lax.dot_general(onehot, cbhi_ref[...], (((1,), (0,)), ((), ())),
                             preferred_element_type=jnp.float32)
         + jax.lax.dot_general(onehot, cblo_ref[...], (((1,), (0,)), ((), ())),
                               preferred_element_type=jnp.float32))

    q_ref[...] = q
    ids_ref[...] = ids.reshape(1, 1, BLK)
    loss_ref[...] = (q - z) ** 2


@functools.partial(jax.jit, static_argnames=())
def kernel(input, codebook):
    B, T, _ = input.shape
    n_tok = B * T
    n_blk = n_tok // BLK
    z = input.reshape(n_tok, D)

    q, ids3, loss = pl.pallas_call(
        _vq_block_kernel,
        grid=(n_blk,),
        in_specs=[
            pl.BlockSpec((BLK, D), lambda i: (i, 0)),
            pl.BlockSpec((K, D), lambda i: (0, 0)),
        ],
        out_specs=[
            pl.BlockSpec((BLK, D), lambda i: (i, 0)),
            pl.BlockSpec((1, 1, BLK), lambda i: (i, 0, 0)),
            pl.BlockSpec((BLK, D), lambda i: (i, 0)),
        ],
        out_shape=[
            jax.ShapeDtypeStruct((n_tok, D), jnp.float32),
            jax.ShapeDtypeStruct((n_blk, 1, BLK), jnp.int32),
            jax.ShapeDtypeStruct((n_tok, D), jnp.float32),
        ],
        scratch_shapes=[
            pltpu.VMEM((1, K), jnp.float32),
            pltpu.VMEM((K, D), jnp.bfloat16),
            pltpu.VMEM((K, D), jnp.bfloat16),
        ],
    )(z, codebook)

    q = q.reshape(B, T, D)
    ids = ids3.reshape(B, T)
    loss = loss.reshape(B, T, D)
    return (q, ids, loss, loss)


# speedup vs baseline: 1.2210x; 1.0210x over previous
import functools

import jax
import jax.numpy as jnp
from jax.experimental import pallas as pl
from jax.experimental.pallas import tpu as pltpu

K = 2048
D = 256
BLK = 2048


def _vq_block_kernel(z_ref, cb_ref, q_ref, ids_ref, loss_ref,
                     c2_ref, cbhi_ref, cblo_ref):
    @pl.when(pl.program_id(0) == 0)
    def _prologue():
        cb0 = cb_ref[...]
        c2_ref[...] = jnp.sum(cb0 * cb0, axis=1)[None, :]
        hi = cb0.astype(jnp.bfloat16)
        cbhi_ref[...] = hi
        cblo_ref[...] = (cb0 - hi.astype(jnp.float32)).astype(jnp.bfloat16)

    z = z_ref[...]
    cb = cb_ref[...]

    mm = jax.lax.dot_general(
        z, cb, (((1,), (1,)), ((), ())),
        preferred_element_type=jnp.float32,
    )
    z2 = jnp.sum(z * z, axis=1, keepdims=True)
    dist = (z2 - 2.0 * mm) + c2_ref[...]

    minval = jnp.min(dist, axis=1, keepdims=True)
    iota = jax.lax.broadcasted_iota(jnp.int32, (BLK, K), 1)
    ids = jnp.min(jnp.where(dist == minval, iota, K), axis=1)

    onehot = (iota == ids[:, None]).astype(jnp.bfloat16)
    q = (jax.lax.dot_general(onehot, cbhi_ref[...], (((1,), (0,)), ((), ())),
                             preferred_element_type=jnp.float32)
         + jax.lax.dot_general(onehot, cblo_ref[...], (((1,), (0,)), ((), ())),
                               preferred_element_type=jnp.float32))

    q_ref[...] = q
    ids_ref[...] = ids.reshape(1, 1, BLK)
    loss_ref[...] = (q - z) ** 2


@functools.partial(jax.jit, static_argnames=())
def kernel(input, codebook):
    B, T, _ = input.shape
    n_tok = B * T
    n_blk = n_tok // BLK
    z = input.reshape(n_tok, D)

    q, ids3, loss = pl.pallas_call(
        _vq_block_kernel,
        grid=(n_blk,),
        in_specs=[
            pl.BlockSpec((BLK, D), lambda i: (i, 0)),
            pl.BlockSpec((K, D), lambda i: (0, 0)),
        ],
        out_specs=[
            pl.BlockSpec((BLK, D), lambda i: (i, 0)),
            pl.BlockSpec((1, 1, BLK), lambda i: (i, 0, 0)),
            pl.BlockSpec((BLK, D), lambda i: (i, 0)),
        ],
        out_shape=[
            jax.ShapeDtypeStruct((n_tok, D), jnp.float32),
            jax.ShapeDtypeStruct((n_blk, 1, BLK), jnp.int32),
            jax.ShapeDtypeStruct((n_tok, D), jnp.float32),
        ],
        scratch_shapes=[
            pltpu.VMEM((1, K), jnp.float32),
            pltpu.VMEM((K, D), jnp.bfloat16),
            pltpu.VMEM((K, D), jnp.bfloat16),
        ],
    )(z, codebook)

    q = q.reshape(B, T, D)
    ids = ids3.reshape(B, T)
    loss = loss.reshape(B, T, D)
    return (q, ids, loss, loss)
